# Initial kernel scaffold; baseline (speedup 1.0000x reference)
import jax
import jax.numpy as jnp
from jax.experimental import pallas as pl


def kernel(z_t, h_t, mem_bank, Wq, bq, Wc, bc, Ws, bs, Wm, bm, Wg1, bg1, Wg2, bg2):
    raise NotImplementedError("write your pallas kernel here")


# speedup vs baseline: 2.7276x; 2.7276x over previous
import jax
import jax.numpy as jnp
from jax.experimental import pallas as pl
from jax.experimental.pallas import tpu as pltpu

D = 512
N = 100000
K = 8
BLK = 2000
NB = N // BLK
NCAND = 7
NEG = -3.0e38


def _sims_body(z_ref, mem_ref, out_ref):
    blk = mem_ref[...]
    z = z_ref[...]
    dot = jax.lax.dot_general(
        z, blk, (((1,), (1,)), ((), ())),
        preferred_element_type=jnp.float32)
    ones = jnp.ones((1, D), jnp.float32)
    nsq = jax.lax.dot_general(
        ones, blk * blk, (((1,), (1,)), ((), ())),
        preferred_element_type=jnp.float32)
    out_ref[0] = dot / (jnp.sqrt(nsq) + 1e-12)


def _topk_body(s_ref, idx_ref):
    s = s_ref[...]
    r = jax.lax.broadcasted_iota(jnp.int32, (NB, BLK), 0)
    c = jax.lax.broadcasted_iota(jnp.int32, (NB, BLK), 1)
    flat = r * BLK + c
    big = jnp.int32(2147483647)
    for k in range(K):
        v = jnp.max(s)
        fi = jnp.min(jnp.where(s == v, flat, big))
        idx_ref[k] = fi
        s = jnp.where(flat == fi, NEG, s)


def _tail_body(idx_ref, mem_grp, z, h, eps, Wq, bq, Wc, bc, Ws,
               Wm, bm, Wg1, bg1, Wg2r, bg2, s_out, alpha_out, protos):
    i = pl.program_id(0)
    sub = idx_ref[i] % 8
    protos[pl.ds(i, 1), :] = mem_grp[0, pl.ds(sub, 1), :]

    @pl.when(i == K - 1)
    def _():
        P = protos[...]
        mean8 = jnp.mean(P, axis=0, keepdims=True)
        mean2 = jnp.mean(P[:2], axis=0, keepdims=True)
        mean3 = jnp.mean(P[:3], axis=0, keepdims=True)
        dreams = jnp.clip(z[...] + eps[...], -2.0, 2.0)
        C = jnp.concatenate(
            [mean8, mean2, mean3, dreams, jnp.zeros((1, D), jnp.float32)],
            axis=0)

        def mm(a, b):
            return jax.lax.dot_general(
                a, b, (((1,), (0,)), ((), ())),
                preferred_element_type=jnp.float32)

        qv = jnp.tanh(mm(z[...], Wq[:D, :]) + mm(h[...], Wq[D:, :])
                      + bq[...])
        A = jnp.tanh(mm(C, Wc[...]) + bc[...])
        w = qv * Ws[...]
        scores = jax.lax.dot_general(
            A, w, (((1,), (1,)), ((), ())),
            preferred_element_type=jnp.float32)
        rows = jax.lax.broadcasted_iota(jnp.int32, (K, 1), 0)
        scores = jnp.where(rows < NCAND, scores, NEG)
        m = jnp.max(scores)
        e = jnp.exp(scores - m)
        alpha = e / jnp.sum(e)
        mix = jax.lax.dot_general(
            alpha, C, (((0,), (0,)), ((), ())),
            preferred_element_type=jnp.float32)
        raw = jnp.tanh(mm(mix, Wm[...]) + bm[...])
        g1 = jnp.tanh(mm(z[...], Wg1[:D, :]) + mm(h[...], Wg1[D:, :])
                      + bg1[...])
        gl = jnp.sum(g1 * Wg2r[...]) + bg2[0]
        gate = jax.nn.sigmoid(gl)
        s_out[...] = gate * raw
        alpha_out[...] = alpha


def kernel(z_t, h_t, mem_bank, Wq, bq, Wc, bc, Ws, bs, Wm, bm,
           Wg1, bg1, Wg2, bg2):
    z2 = z_t.reshape(1, D)
    h2 = h_t.reshape(1, D)

    sims = pl.pallas_call(
        _sims_body,
        grid=(NB,),
        in_specs=[
            pl.BlockSpec((1, D), lambda i: (0, 0)),
            pl.BlockSpec((BLK, D), lambda i: (i, 0)),
        ],
        out_specs=pl.BlockSpec((1, 1, BLK), lambda i: (i, 0, 0)),
        out_shape=jax.ShapeDtypeStruct((NB, 1, BLK), jnp.float32),
        compiler_params=pltpu.CompilerParams(
            dimension_semantics=("arbitrary",)),
    )(z2, mem_bank)

    idx = pl.pallas_call(
        _topk_body,
        in_specs=[pl.BlockSpec((NB, BLK), lambda: (0, 0))],
        out_specs=pl.BlockSpec(memory_space=pltpu.SMEM),
        out_shape=jax.ShapeDtypeStruct((K,), jnp.int32),
    )(sims.reshape(NB, BLK))

    eps = 0.08 * jax.random.normal(jax.random.key(1), (4, D), jnp.float32)

    grid_spec = pltpu.PrefetchScalarGridSpec(
        num_scalar_prefetch=1,
        grid=(K,),
        in_specs=[
            pl.BlockSpec((1, 8, D), lambda i, idx_ref: (idx_ref[i] // 8, 0, 0)),
            pl.BlockSpec((1, D), lambda i, idx_ref: (0, 0)),
            pl.BlockSpec((1, D), lambda i, idx_ref: (0, 0)),
            pl.BlockSpec((4, D), lambda i, idx_ref: (0, 0)),
            pl.BlockSpec((2 * D, D), lambda i, idx_ref: (0, 0)),
            pl.BlockSpec((1, D), lambda i, idx_ref: (0, 0)),
            pl.BlockSpec((D, D), lambda i, idx_ref: (0, 0)),
            pl.BlockSpec((1, D), lambda i, idx_ref: (0, 0)),
            pl.BlockSpec((1, D), lambda i, idx_ref: (0, 0)),
            pl.BlockSpec((D, D), lambda i, idx_ref: (0, 0)),
            pl.BlockSpec((1, D), lambda i, idx_ref: (0, 0)),
            pl.BlockSpec((2 * D, D), lambda i, idx_ref: (0, 0)),
            pl.BlockSpec((1, D), lambda i, idx_ref: (0, 0)),
            pl.BlockSpec((1, D), lambda i, idx_ref: (0, 0)),
            pl.BlockSpec(memory_space=pltpu.SMEM),
        ],
        out_specs=[
            pl.BlockSpec((1, D), lambda i, idx_ref: (0, 0)),
            pl.BlockSpec((K, 1), lambda i, idx_ref: (0, 0)),
        ],
        scratch_shapes=[pltpu.VMEM((K, D), jnp.float32)],
    )

    s2, alpha8 = pl.pallas_call(
        _tail_body,
        grid_spec=grid_spec,
        out_shape=[
            jax.ShapeDtypeStruct((1, D), jnp.float32),
            jax.ShapeDtypeStruct((K, 1), jnp.float32),
        ],
    )(idx, mem_bank.reshape(N // 8, 8, D), z2, h2, eps,
      Wq, bq.reshape(1, D), Wc, bc.reshape(1, D), Ws.reshape(1, D),
      Wm, bm.reshape(1, D), Wg1, bg1.reshape(1, D),
      Wg2.reshape(1, D), bg2.reshape(1))

    return (s2.reshape(D), alpha8[:NCAND, 0])
